# baseline (speedup 1.0000x reference)
import functools

import jax
import jax.numpy as jnp
from jax import lax
from jax.experimental import pallas as pl
from jax.experimental.pallas import tpu as pltpu

_K_DPC = 16
_NUM_CENTROIDS = 50


def _dpc_body(x_ref, noise_ref, out_ref, d2_s, *, n, c, rt):
    xf = x_ref[0]
    xsq = jnp.sum(xf * xf, axis=1)
    xsq_col = jnp.sum(xf * xf, axis=1, keepdims=True)
    inv_sqrt_c = 1.0 / (c ** 0.5)
    ntiles = n // rt

    d2max = jnp.float32(-jnp.inf)
    m1_parts = []
    for t in range(ntiles):
        r0 = t * rt
        xt = xf[r0:r0 + rt, :]
        g = lax.dot_general(
            xf, xt, (((1,), (1,)), ((), ())),
            preferred_element_type=jnp.float32,
            precision=lax.Precision.DEFAULT,
        )
        d2t = (xsq_col + xsq[r0:r0 + rt][None, :]) - 2.0 * g
        d2t = jnp.maximum(d2t, 1e-12)
        d2_s[:, pl.ds(r0, rt)] = d2t
        d2max = jnp.maximum(d2max, jnp.max(d2t))
        m1_parts.append(jnp.min(d2t, axis=0))

    def tile_step(d2t, st):
        prev_m, prev_a, rem, acc = st
        gt = d2t > prev_m[None, :]
        a = jnp.sum(gt.astype(jnp.float32), axis=0)
        take = jnp.minimum(prev_a - a, rem)
        md = jnp.sqrt(prev_m) * inv_sqrt_c
        acc = acc + jnp.where(take > 0.0, take * (md * md), 0.0)
        m = jnp.min(jnp.where(gt, d2t, jnp.inf), axis=0)
        return m, a, rem - take, acc

    def step(_, carry):
        return tuple(
            tile_step(d2_s[:, pl.ds(t * rt, rt)], carry[t])
            for t in range(ntiles)
        )

    carry0 = tuple(
        (
            m1_parts[t],
            jnp.full((rt,), float(n), jnp.float32),
            jnp.full((rt,), float(_K_DPC), jnp.float32),
            jnp.zeros((rt,), jnp.float32),
        )
        for t in range(ntiles)
    )
    carry = lax.fori_loop(0, _K_DPC - 1, step, carry0)

    dens_parts = []
    for t in range(ntiles):
        m, a, rem, acc = carry[t]
        d2t = d2_s[:, pl.ds(t * rt, rt)]
        a_last = jnp.sum((d2t > m[None, :]).astype(jnp.float32), axis=0)
        take = jnp.minimum(a - a_last, rem)
        md = jnp.sqrt(m) * inv_sqrt_c
        acc = acc + jnp.where(take > 0.0, take * (md * md), 0.0)
        dens_parts.append(jnp.exp(-(acc / _K_DPC)))
    dens = jnp.concatenate(dens_parts) + noise_ref[0, 0]

    dens_col = dens[:, None]
    score_parts = []
    for t in range(ntiles):
        r0 = t * rt
        d2t = d2_s[:, pl.ds(r0, rt)]
        dens_t = dens[r0:r0 + rt]
        masked_pd = jnp.where(dens_col > dens_t[None, :], d2t, d2max)
        pd2 = jnp.min(masked_pd, axis=0)
        score_parts.append((jnp.sqrt(pd2) / (c ** 0.5)) * dens_t)
    score = jnp.concatenate(score_parts)

    score_col = score[:, None]
    rank_parts = []
    for t in range(ntiles):
        r0 = t * rt
        st = score[r0:r0 + rt]
        jj = lax.broadcasted_iota(jnp.int32, (n, rt), 0)
        ii = lax.broadcasted_iota(jnp.int32, (n, rt), 1) + r0
        gt = (score_col > st[None, :]).astype(jnp.int32)
        eq_lower = ((score_col == st[None, :]) & (jj < ii)).astype(jnp.int32)
        rank_parts.append(jnp.sum(gt + eq_lower, axis=0))
    rank = jnp.concatenate(rank_parts)

    srank = lax.broadcasted_iota(jnp.int32, (_NUM_CENTROIDS, n), 0)
    onehot = (rank[None, :] == srank).astype(jnp.float32)
    centers = lax.dot_general(
        xf, onehot, (((0,), (1,)), ((), ())),
        preferred_element_type=jnp.float32,
        precision=lax.Precision.HIGHEST,
    )
    out_ref[0] = centers


def kernel(x, relative_pos, num_centroids):
    b, c, h, w = x.shape
    n = h * w
    xf = jnp.reshape(x, (b, n, c))
    noise = jax.random.uniform(jax.random.key(1), (b, n), dtype=jnp.float32) * 1e-6
    noise = jnp.reshape(noise, (b, 1, n))
    body = functools.partial(_dpc_body, n=n, c=c, rt=min(n, 512))
    out = pl.pallas_call(
        body,
        grid=(b,),
        in_specs=[
            pl.BlockSpec((1, n, c), lambda i: (i, 0, 0)),
            pl.BlockSpec((1, 1, n), lambda i: (i, 0, 0)),
        ],
        out_specs=pl.BlockSpec((1, c, _NUM_CENTROIDS), lambda i: (i, 0, 0)),
        out_shape=jax.ShapeDtypeStruct((b, c, _NUM_CENTROIDS), jnp.float32),
        scratch_shapes=[pltpu.VMEM((n, n), jnp.float32)],
        compiler_params=pltpu.CompilerParams(
            dimension_semantics=("arbitrary",),
        ),
    )(xf, noise)
    return out


# speedup vs baseline: 1.3051x; 1.3051x over previous
import functools

import jax
import jax.numpy as jnp
from jax import lax
from jax.experimental import pallas as pl
from jax.experimental.pallas import tpu as pltpu

_K_DPC = 16
_NUM_CENTROIDS = 50


def _dpc_body(x_ref, noise_ref, out_ref, d2_s, *, n, c, rt):
    xf = x_ref[0]
    xsq = jnp.sum(xf * xf, axis=1)
    xsq_col = jnp.sum(xf * xf, axis=1, keepdims=True)
    inv_sqrt_c = 1.0 / (c ** 0.5)
    ntiles = n // rt

    d2max = jnp.float32(-jnp.inf)
    m1_parts = []
    for t in range(ntiles):
        r0 = t * rt
        xt = xf[r0:r0 + rt, :]
        g = lax.dot_general(
            xf, xt, (((1,), (1,)), ((), ())),
            preferred_element_type=jnp.float32,
            precision=lax.Precision.DEFAULT,
        )
        d2t = (xsq_col + xsq[r0:r0 + rt][None, :]) - 2.0 * g
        d2t = jnp.maximum(d2t, 1e-12)
        d2_s[:, pl.ds(r0, rt)] = d2t
        d2max = jnp.maximum(d2max, jnp.max(d2t))
        m1_parts.append(jnp.min(d2t, axis=0))

    def tile_step(d2t, st):
        prev_m, prev_a, rem, acc = st
        gt = d2t > prev_m[None, :]
        a = jnp.sum(gt.astype(jnp.float32), axis=0)
        take = jnp.minimum(prev_a - a, rem)
        md = jnp.sqrt(prev_m) * inv_sqrt_c
        acc = acc + jnp.where(take > 0.0, take * (md * md), 0.0)
        m = jnp.min(jnp.where(gt, d2t, jnp.inf), axis=0)
        return m, a, rem - take, acc

    def step(_, carry):
        return tuple(
            tile_step(d2_s[:, pl.ds(t * rt, rt)], carry[t])
            for t in range(ntiles)
        )

    carry0 = tuple(
        (
            m1_parts[t],
            jnp.full((rt,), float(n), jnp.float32),
            jnp.full((rt,), float(_K_DPC), jnp.float32),
            jnp.zeros((rt,), jnp.float32),
        )
        for t in range(ntiles)
    )
    carry = lax.fori_loop(0, _K_DPC - 1, step, carry0)

    dens_parts = []
    for t in range(ntiles):
        m, a, rem, acc = carry[t]
        d2t = d2_s[:, pl.ds(t * rt, rt)]
        a_last = jnp.sum((d2t > m[None, :]).astype(jnp.float32), axis=0)
        take = jnp.minimum(a - a_last, rem)
        md = jnp.sqrt(m) * inv_sqrt_c
        acc = acc + jnp.where(take > 0.0, take * (md * md), 0.0)
        dens_parts.append(jnp.exp(-(acc / _K_DPC)))
    dens = jnp.concatenate(dens_parts) + noise_ref[0, 0]

    dens_col = dens[:, None]
    score_parts = []
    for t in range(ntiles):
        r0 = t * rt
        d2t = d2_s[:, pl.ds(r0, rt)]
        dens_t = dens[r0:r0 + rt]
        masked_pd = jnp.where(dens_col > dens_t[None, :], d2t, d2max)
        pd2 = jnp.min(masked_pd, axis=0)
        score_parts.append((jnp.sqrt(pd2) / (c ** 0.5)) * dens_t)
    score = jnp.concatenate(score_parts)

    score_col = score[:, None]
    rank_parts = []
    for t in range(ntiles):
        r0 = t * rt
        st = score[r0:r0 + rt]
        jj = lax.broadcasted_iota(jnp.int32, (n, rt), 0)
        ii = lax.broadcasted_iota(jnp.int32, (n, rt), 1) + r0
        gt = (score_col > st[None, :]).astype(jnp.int32)
        eq_lower = ((score_col == st[None, :]) & (jj < ii)).astype(jnp.int32)
        rank_parts.append(jnp.sum(gt + eq_lower, axis=0))
    rank = jnp.concatenate(rank_parts)

    srank = lax.broadcasted_iota(jnp.int32, (_NUM_CENTROIDS, n), 0)
    onehot = (rank[None, :] == srank).astype(jnp.float32)
    centers = lax.dot_general(
        xf, onehot, (((0,), (1,)), ((), ())),
        preferred_element_type=jnp.float32,
        precision=lax.Precision.HIGHEST,
    )
    out_ref[0] = centers


def kernel(x, relative_pos, num_centroids):
    b, c, h, w = x.shape
    n = h * w
    xf = jnp.reshape(x, (b, n, c))
    noise = jax.random.uniform(jax.random.key(1), (b, n), dtype=jnp.float32) * 1e-6
    noise = jnp.reshape(noise, (b, 1, n))
    body = functools.partial(_dpc_body, n=n, c=c, rt=min(n, 256))
    out = pl.pallas_call(
        body,
        grid=(b,),
        in_specs=[
            pl.BlockSpec((1, n, c), lambda i: (i, 0, 0)),
            pl.BlockSpec((1, 1, n), lambda i: (i, 0, 0)),
        ],
        out_specs=pl.BlockSpec((1, c, _NUM_CENTROIDS), lambda i: (i, 0, 0)),
        out_shape=jax.ShapeDtypeStruct((b, c, _NUM_CENTROIDS), jnp.float32),
        scratch_shapes=[pltpu.VMEM((n, n), jnp.float32)],
        compiler_params=pltpu.CompilerParams(
            dimension_semantics=("arbitrary",),
        ),
    )(xf, noise)
    return out
